# baseline (speedup 1.0000x reference)
import functools

import jax
import jax.numpy as jnp
from jax import lax
from jax.experimental import pallas as pl
from jax.experimental.pallas import tpu as pltpu
from jax.experimental.pallas import tpu_sc as plsc

B, S, D = 4, 8192, 768
NC, NS = 2, 16
NW = NC * NS
POS_PER_W = S // NW
CH = 8
NCHUNK = POS_PER_W // CH
STEPS = NCHUNK * B
NXB = 12

_mesh = plsc.VectorSubcoreMesh(core_axis_name="c", subcore_axis_name="s")


@functools.partial(
    pl.kernel,
    out_type=jax.ShapeDtypeStruct((B, S, D), jnp.float32),
    mesh=_mesh,
    compiler_params=pltpu.CompilerParams(use_tc_tiling_on_sc=True),
    scratch_types=[
        pltpu.VMEM((CH, D), jnp.float32),
        pltpu.VMEM((CH, D), jnp.float32),
        pltpu.VMEM((CH, D), jnp.float32),
        pltpu.VMEM((CH, D), jnp.float32),
        pltpu.VMEM((CH, D), jnp.float32),
        pltpu.VMEM((CH, D), jnp.float32),
        pltpu.VMEM((CH, D), jnp.float32),
        pltpu.VMEM((CH, D), jnp.float32),
        pltpu.VMEM((CH, D), jnp.float32),
        pltpu.VMEM((CH, D), jnp.float32),
        pltpu.VMEM((CH, D), jnp.float32),
        pltpu.VMEM((CH, D), jnp.float32),
        pltpu.VMEM((CH, D), jnp.float32),
        pltpu.VMEM((CH, D), jnp.float32),
        pltpu.SemaphoreType.DMA,
        pltpu.SemaphoreType.DMA,
        pltpu.SemaphoreType.DMA,
        pltpu.SemaphoreType.DMA,
        pltpu.SemaphoreType.DMA,
        pltpu.SemaphoreType.DMA,
        pltpu.SemaphoreType.DMA,
        pltpu.SemaphoreType.DMA,
        pltpu.SemaphoreType.DMA,
        pltpu.SemaphoreType.DMA,
        pltpu.SemaphoreType.DMA,
        pltpu.SemaphoreType.DMA,
        pltpu.SemaphoreType.DMA,
        pltpu.SemaphoreType.DMA,
        pltpu.SemaphoreType.DMA,
        pltpu.SemaphoreType.DMA,
        pltpu.SemaphoreType.DMA,
        pltpu.SemaphoreType.DMA,
        pltpu.SemaphoreType.DMA,
        pltpu.SemaphoreType.DMA,
        pltpu.SemaphoreType.DMA,
        pltpu.SemaphoreType.DMA,
        pltpu.SemaphoreType.DMA,
        pltpu.SemaphoreType.DMA,
        pltpu.SemaphoreType.DMA,
        pltpu.SemaphoreType.DMA,
    ],
)
def _sc_add(x_hbm, pos_hbm, out_hbm, pv0, pv1,
            xv0, xv1, xv2, xv3, xv4, xv5, xv6, xv7, xv8, xv9, xv10, xv11,
            ps0, ps1, is0, is1, is2, is3, is4, is5, is6, is7, is8, is9,
            is10, is11,
            os0, os1, os2, os3, os4, os5, os6, os7, os8, os9, os10, os11):
    wid = lax.axis_index("s") * NC + lax.axis_index("c")
    row_base = wid * POS_PER_W
    pbufs, psems = (pv0, pv1), (ps0, ps1)
    xbufs = (xv0, xv1, xv2, xv3, xv4, xv5, xv6, xv7, xv8, xv9, xv10, xv11)
    isems = (is0, is1, is2, is3, is4, is5, is6, is7, is8, is9, is10, is11)
    osems = (os0, os1, os2, os3, os4, os5, os6, os7, os8, os9, os10, os11)

    def rows(i):
        c, b = divmod(i, B)
        return b, row_base + c * CH

    def x_in(i):
        b, r0 = rows(i)
        return pltpu.async_copy(
            x_hbm.at[b, pl.ds(r0, CH), :], xbufs[i % NXB], isems[i % NXB])

    pos_d = {0: pltpu.async_copy(
        pos_hbm.at[pl.ds(row_base, CH), :], pv0, ps0)}
    in_d = {i: x_in(i) for i in range(8)}
    out_d = {}

    for i in range(STEPS):
        c, b = divmod(i, B)
        buf = xbufs[i % NXB]
        pbuf = pbufs[c % 2]
        if b == 0:
            pos_d[c].wait()
            if c + 1 < NCHUNK:
                pos_d[c + 1] = pltpu.async_copy(
                    pos_hbm.at[pl.ds(row_base + (c + 1) * CH, CH), :],
                    pbufs[(c + 1) % 2], psems[(c + 1) % 2])
        if i + 8 < STEPS:
            if i - 3 >= 0:
                out_d[i - 3].wait()
            in_d[i + 8] = x_in(i + 8)
        in_d[i].wait()

        @plsc.parallel_loop(0, CH, 1)
        def _(r):
            @plsc.parallel_loop(0, D, 16, unroll=8)
            def _(j):
                plsc.addupdate(buf.at[r, pl.ds(j, 16)],
                               pbuf[r, pl.ds(j, 16)])

        b_i, r0_i = rows(i)
        out_d[i] = pltpu.async_copy(
            buf, out_hbm.at[b_i, pl.ds(r0_i, CH), :], osems[i % NXB])

    for k in range(8):
        out_d[STEPS - 8 + k].wait()


def kernel(x, pos_table):
    Bx, Sx, Dx = x.shape
    return _sc_add(x, pos_table[:Sx])


# speedup vs baseline: 1.0259x; 1.0259x over previous
import functools

import jax
import jax.numpy as jnp
from jax import lax
from jax.experimental import pallas as pl
from jax.experimental.pallas import tpu as pltpu
from jax.experimental.pallas import tpu_sc as plsc

B, S, D = 4, 8192, 768
NC, NS = 2, 16
NW = NC * NS
POS_PER_W = S // NW
CH = 16
NCHUNK = POS_PER_W // CH
STEPS = NCHUNK * B
NXB = 6
AHEAD = 4

_mesh = plsc.VectorSubcoreMesh(core_axis_name="c", subcore_axis_name="s")


@functools.partial(
    pl.kernel,
    out_type=jax.ShapeDtypeStruct((B, S, D), jnp.float32),
    mesh=_mesh,
    compiler_params=pltpu.CompilerParams(use_tc_tiling_on_sc=True),
    scratch_types=[
        pltpu.VMEM((CH, D), jnp.float32),
        pltpu.VMEM((CH, D), jnp.float32),
        pltpu.VMEM((CH, D), jnp.float32),
        pltpu.VMEM((CH, D), jnp.float32),
        pltpu.VMEM((CH, D), jnp.float32),
        pltpu.VMEM((CH, D), jnp.float32),
        pltpu.VMEM((CH, D), jnp.float32),
        pltpu.VMEM((CH, D), jnp.float32),
        pltpu.SemaphoreType.DMA,
        pltpu.SemaphoreType.DMA,
        pltpu.SemaphoreType.DMA,
        pltpu.SemaphoreType.DMA,
        pltpu.SemaphoreType.DMA,
        pltpu.SemaphoreType.DMA,
        pltpu.SemaphoreType.DMA,
        pltpu.SemaphoreType.DMA,
        pltpu.SemaphoreType.DMA,
        pltpu.SemaphoreType.DMA,
        pltpu.SemaphoreType.DMA,
        pltpu.SemaphoreType.DMA,
        pltpu.SemaphoreType.DMA,
        pltpu.SemaphoreType.DMA,
    ],
)
def _sc_add(x_hbm, pos_hbm, out_hbm, pv0, pv1,
            xv0, xv1, xv2, xv3, xv4, xv5,
            ps0, ps1, is0, is1, is2, is3, is4, is5,
            os0, os1, os2, os3, os4, os5):
    wid = lax.axis_index("s") * NC + lax.axis_index("c")
    row_base = wid * POS_PER_W
    pbufs, psems = (pv0, pv1), (ps0, ps1)
    xbufs = (xv0, xv1, xv2, xv3, xv4, xv5)
    isems = (is0, is1, is2, is3, is4, is5)
    osems = (os0, os1, os2, os3, os4, os5)

    def rows(i):
        c, b = divmod(i, B)
        return b, row_base + c * CH

    def x_in(i):
        b, r0 = rows(i)
        return pltpu.async_copy(
            x_hbm.at[b, pl.ds(r0, CH), :], xbufs[i % NXB], isems[i % NXB])

    pos_d = {0: pltpu.async_copy(
        pos_hbm.at[pl.ds(row_base, CH), :], pv0, ps0)}
    in_d = {i: x_in(i) for i in range(AHEAD)}
    out_d = {}

    for i in range(STEPS):
        c, b = divmod(i, B)
        buf = xbufs[i % NXB]
        pbuf = pbufs[c % 2]
        if b == 0:
            pos_d[c].wait()
            if c + 1 < NCHUNK:
                pos_d[c + 1] = pltpu.async_copy(
                    pos_hbm.at[pl.ds(row_base + (c + 1) * CH, CH), :],
                    pbufs[(c + 1) % 2], psems[(c + 1) % 2])
        if i + AHEAD < STEPS:
            if i - 2 >= 0:
                out_d[i - 2].wait()
            in_d[i + AHEAD] = x_in(i + AHEAD)
        in_d[i].wait()

        @plsc.parallel_loop(0, CH, 1)
        def _(r):
            @plsc.parallel_loop(0, D, 16, unroll=8)
            def _(j):
                plsc.addupdate(buf.at[r, pl.ds(j, 16)],
                               pbuf[r, pl.ds(j, 16)])

        b_i, r0_i = rows(i)
        out_d[i] = pltpu.async_copy(
            buf, out_hbm.at[b_i, pl.ds(r0_i, CH), :], osems[i % NXB])

    for k in range(AHEAD):
        out_d[STEPS - AHEAD + k].wait()


def kernel(x, pos_table):
    Bx, Sx, Dx = x.shape
    return _sc_add(x, pos_table[:Sx])
